# baseline (speedup 1.0000x reference)
import functools
import math

import jax
import jax.numpy as jnp
from jax import lax
from jax.experimental import pallas as pl
from jax.experimental.pallas import tpu as pltpu
from jax.experimental.pallas import tpu_sc as plsc

N = 10000
E = 320000
RATIO = 0.5
K1 = int(math.ceil(RATIO * N))
K2 = int(math.ceil(RATIO * K1))
K3 = int(math.ceil(RATIO * K2))

NC = 2
NS = 16
NW = NC * NS
EPW = E // NW
L = 16


def _mm_relu_body(x_ref, w_ref, b_ref, ws_ref, xh_ref, h_ref):
    xh = jnp.dot(x_ref[...], w_ref[...], preferred_element_type=jnp.float32)
    xh = jnp.maximum(xh + b_ref[...], 0.0)
    xh_ref[...] = xh
    h_ref[...] = jnp.dot(xh, ws_ref[...], preferred_element_type=jnp.float32)


def mm_relu_h(x, W, b, Ws, block):
    n, d = x.shape
    hdim = W.shape[1]
    return pl.pallas_call(
        _mm_relu_body,
        grid=(n // block,),
        in_specs=[
            pl.BlockSpec((block, d), lambda i: (i, 0)),
            pl.BlockSpec((d, hdim), lambda i: (0, 0)),
            pl.BlockSpec((hdim,), lambda i: (0,)),
            pl.BlockSpec((hdim, 1), lambda i: (0, 0)),
        ],
        out_specs=[
            pl.BlockSpec((block, hdim), lambda i: (i, 0)),
            pl.BlockSpec((block, 1), lambda i: (i, 0)),
        ],
        out_shape=[
            jax.ShapeDtypeStruct((n, hdim), jnp.float32),
            jax.ShapeDtypeStruct((n, 1), jnp.float32),
        ],
    )(x, W, b, Ws)


def _ew_body(fn, x_ref, o_ref):
    o_ref[...] = fn(x_ref[...])


def _ew1d(fn, x):
    return pl.pallas_call(
        functools.partial(_ew_body, fn),
        out_shape=jax.ShapeDtypeStruct(x.shape, x.dtype),
    )(x)


def p_dinv(deg_raw):
    return _ew1d(lambda v: lax.rsqrt(v + 1.0), deg_raw)


def p_tanh(x):
    return _ew1d(jnp.tanh, x)


def _wid():
    return lax.axis_index("s") * NC + lax.axis_index("c")


def _remap_body(src_hbm, dst_hbm, w_hbm, imap_hbm, kept_hbm,
                srcn_hbm, dstn_hbm, wn_hbm,
                imap_v, kept_v, src_v, dst_v, w_v, srcn_v, dstn_v, wn_v):
    base = _wid() * EPW
    pltpu.sync_copy(imap_hbm, imap_v)
    pltpu.sync_copy(kept_hbm, kept_v)
    pltpu.sync_copy(src_hbm.at[pl.ds(base, EPW)], src_v)
    pltpu.sync_copy(dst_hbm.at[pl.ds(base, EPW)], dst_v)
    pltpu.sync_copy(w_hbm.at[pl.ds(base, EPW)], w_v)

    def step(i, carry):
        o = i * L
        s16 = src_v[pl.ds(o, L)]
        d16 = dst_v[pl.ds(o, L)]
        w16 = w_v[pl.ds(o, L)]
        ks = plsc.load_gather(kept_v, [s16])
        kd = plsc.load_gather(kept_v, [d16])
        srcn_v[pl.ds(o, L)] = plsc.load_gather(imap_v, [s16])
        dstn_v[pl.ds(o, L)] = plsc.load_gather(imap_v, [d16])
        wn_v[pl.ds(o, L)] = (w16 * ks) * kd
        return carry

    lax.fori_loop(0, EPW // L, step, 0)
    pltpu.sync_copy(srcn_v, srcn_hbm.at[pl.ds(base, EPW)])
    pltpu.sync_copy(dstn_v, dstn_hbm.at[pl.ds(base, EPW)])
    pltpu.sync_copy(wn_v, wn_hbm.at[pl.ds(base, EPW)])


def sc_remap(src, dst, w, imap, kept):
    m = imap.shape[0]
    mesh = plsc.VectorSubcoreMesh(core_axis_name="c", subcore_axis_name="s")
    f = functools.partial(
        pl.kernel,
        mesh=mesh,
        compiler_params=pltpu.CompilerParams(needs_layout_passes=False),
        out_type=[
            jax.ShapeDtypeStruct((E,), jnp.int32),
            jax.ShapeDtypeStruct((E,), jnp.int32),
            jax.ShapeDtypeStruct((E,), jnp.float32),
        ],
        scratch_types=[
            pltpu.VMEM((m,), jnp.int32),
            pltpu.VMEM((m,), jnp.float32),
            pltpu.VMEM((EPW,), jnp.int32),
            pltpu.VMEM((EPW,), jnp.int32),
            pltpu.VMEM((EPW,), jnp.float32),
            pltpu.VMEM((EPW,), jnp.int32),
            pltpu.VMEM((EPW,), jnp.int32),
            pltpu.VMEM((EPW,), jnp.float32),
        ],
    )(_remap_body)
    return f(src, dst, w, imap, kept)


def _edgeval_body(src_hbm, dst_hbm, w_hbm, dinv_hbm, h_hbm,
                  val_hbm,
                  dinv_v, h_v, src_v, dst_v, w_v, val_v):
    base = _wid() * EPW
    pltpu.sync_copy(dinv_hbm, dinv_v)
    pltpu.sync_copy(h_hbm, h_v)
    pltpu.sync_copy(src_hbm.at[pl.ds(base, EPW)], src_v)
    pltpu.sync_copy(dst_hbm.at[pl.ds(base, EPW)], dst_v)
    pltpu.sync_copy(w_hbm.at[pl.ds(base, EPW)], w_v)

    def step(i, carry):
        o = i * L
        s16 = src_v[pl.ds(o, L)]
        d16 = dst_v[pl.ds(o, L)]
        w16 = w_v[pl.ds(o, L)]
        dis = plsc.load_gather(dinv_v, [s16])
        did = plsc.load_gather(dinv_v, [d16])
        hs = plsc.load_gather(h_v, [s16])
        val_v[pl.ds(o, L)] = ((dis * did) * w16) * hs
        return carry

    lax.fori_loop(0, EPW // L, step, 0)
    pltpu.sync_copy(val_v, val_hbm.at[pl.ds(base, EPW)])


def sc_edge_vals(src, dst, w, dinv, h):
    m = dinv.shape[0]
    mesh = plsc.VectorSubcoreMesh(core_axis_name="c", subcore_axis_name="s")
    f = functools.partial(
        pl.kernel,
        mesh=mesh,
        compiler_params=pltpu.CompilerParams(needs_layout_passes=False),
        out_type=jax.ShapeDtypeStruct((E,), jnp.float32),
        scratch_types=[
            pltpu.VMEM((m,), jnp.float32),
            pltpu.VMEM((m,), jnp.float32),
            pltpu.VMEM((EPW,), jnp.int32),
            pltpu.VMEM((EPW,), jnp.int32),
            pltpu.VMEM((EPW,), jnp.float32),
            pltpu.VMEM((EPW,), jnp.float32),
        ],
    )(_edgeval_body)
    return f(src, dst, w, dinv, h)


def _topk_perm(score, k):
    i = lax.bitcast_convert_type(score, jnp.int32)
    key = jnp.where(i < 0, i ^ jnp.int32(0x7FFFFFFF), i)
    order = jnp.lexsort((jnp.arange(score.shape[0], dtype=jnp.int32), ~key))
    return order[:k].astype(jnp.int32)


def _score(h, src, dst, w, dinv, b):
    vals = sc_edge_vals(src, dst, w, dinv, h[:, 0])
    agg = jnp.zeros_like(h).at[dst].add(vals[:, None])
    agg = agg + (dinv * dinv)[:, None] * h
    return (agg + b)[:, 0]


def _pool(xh, score, k, n_nodes):
    perm = _topk_perm(score, k)
    x_new = xh[perm] * p_tanh(score[perm])[:, None]
    kept = jnp.zeros((n_nodes,), jnp.float32).at[perm].set(1.0)
    imap = jnp.zeros((n_nodes,), jnp.int32).at[perm].set(
        jnp.arange(k, dtype=jnp.int32))
    return perm, x_new, kept, imap


def _readout(x):
    return jnp.concatenate([jnp.max(x, axis=0), jnp.mean(x, axis=0)])[None, :]


def kernel(x, edge_index, batch, W1, b1, Ws1, bs1, W2, b2, Ws2, bs2, W3, b3, Ws3, bs3, L1W, L1b, L2W, L2b, L3W, L3b):
    src, dst = edge_index[0], edge_index[1]
    w = jnp.ones((E,), jnp.float32)

    xh, h1 = mm_relu_h(x, W1, b1, Ws1, block=1000)
    deg1 = jnp.zeros((N,), jnp.float32).at[dst].add(w)
    dinv1 = p_dinv(deg1)
    score1 = _score(h1, src, dst, w, dinv1, bs1)
    perm1, x_p1, kept1, imap1 = _pool(xh, score1, K1, N)
    x1 = _readout(x_p1)

    src2, dst2, w2 = sc_remap(src, dst, w, imap1, kept1)
    x2h, h2 = mm_relu_h(x_p1, W2, b2, Ws2, block=1000)
    deg2 = jnp.zeros((K1,), jnp.float32).at[dst2].add(w2)
    dinv2 = p_dinv(deg2)
    score2 = _score(h2, src2, dst2, w2, dinv2, bs2)
    perm2, x_p2, kept2, imap2 = _pool(x2h, score2, K2, K1)
    x2 = _readout(x_p2)

    src3, dst3, w3 = sc_remap(src2, dst2, w2, imap2, kept2)
    x3h, h3 = mm_relu_h(x_p2, W3, b3, Ws3, block=2500)
    deg3 = jnp.zeros((K2,), jnp.float32).at[dst3].add(w3)
    dinv3 = p_dinv(deg3)
    score3 = _score(h3, src3, dst3, w3, dinv3, bs3)
    perm3 = _topk_perm(score3, K3)
    x_p3 = x3h[perm3] * p_tanh(score3[perm3])[:, None]
    x3 = _readout(x_p3)

    xo = x1 + x2 + x3

    def head(v):
        v = jax.nn.relu(v @ L1W + L1b)
        v = jax.nn.relu(v @ L2W + L2b)
        return jax.nn.log_softmax(v @ L3W + L3b, axis=-1)

    ho = head(xo)
    return (ho, ho, score1, score1, score2, score2, score3, score3)


# speedup vs baseline: 1.5569x; 1.5569x over previous
import functools
import math

import jax
import jax.numpy as jnp
from jax import lax
from jax.experimental import pallas as pl
from jax.experimental.pallas import tpu as pltpu
from jax.experimental.pallas import tpu_sc as plsc

N = 10000
E = 320000
RATIO = 0.5
K1 = int(math.ceil(RATIO * N))
K2 = int(math.ceil(RATIO * K1))
K3 = int(math.ceil(RATIO * K2))

NC = 2
NS = 16
NW = NC * NS
EPW = E // NW
L = 16


def _mm_relu_body(x_ref, w_ref, b_ref, ws_ref, xh_ref, h_ref):
    xh = jnp.dot(x_ref[...], w_ref[...], preferred_element_type=jnp.float32)
    xh = jnp.maximum(xh + b_ref[...], 0.0)
    xh_ref[...] = xh
    h_ref[...] = jnp.dot(xh, ws_ref[...], preferred_element_type=jnp.float32)


def mm_relu_h(x, W, b, Ws, block):
    n, d = x.shape
    hdim = W.shape[1]
    return pl.pallas_call(
        _mm_relu_body,
        grid=(n // block,),
        in_specs=[
            pl.BlockSpec((block, d), lambda i: (i, 0)),
            pl.BlockSpec((d, hdim), lambda i: (0, 0)),
            pl.BlockSpec((hdim,), lambda i: (0,)),
            pl.BlockSpec((hdim, 1), lambda i: (0, 0)),
        ],
        out_specs=[
            pl.BlockSpec((block, hdim), lambda i: (i, 0)),
            pl.BlockSpec((block, 1), lambda i: (i, 0)),
        ],
        out_shape=[
            jax.ShapeDtypeStruct((n, hdim), jnp.float32),
            jax.ShapeDtypeStruct((n, 1), jnp.float32),
        ],
    )(x, W, b, Ws)


def _ew_body(fn, x_ref, o_ref):
    o_ref[...] = fn(x_ref[...])


def _ew1d(fn, x):
    return pl.pallas_call(
        functools.partial(_ew_body, fn),
        out_shape=jax.ShapeDtypeStruct(x.shape, x.dtype),
    )(x)


def p_dinv(deg_raw):
    return _ew1d(lambda v: lax.rsqrt(v + 1.0), deg_raw)


def p_tanh(x):
    return _ew1d(jnp.tanh, x)


def _wid():
    return lax.axis_index("s") * NC + lax.axis_index("c")


def _remap_body(src_hbm, dst_hbm, w_hbm, imap_hbm, kept_hbm,
                srcn_hbm, dstn_hbm, wn_hbm,
                imap_v, kept_v, src_v, dst_v, w_v, srcn_v, dstn_v, wn_v):
    base = _wid() * EPW
    pltpu.sync_copy(imap_hbm, imap_v)
    pltpu.sync_copy(kept_hbm, kept_v)
    pltpu.sync_copy(src_hbm.at[pl.ds(base, EPW)], src_v)
    pltpu.sync_copy(dst_hbm.at[pl.ds(base, EPW)], dst_v)
    pltpu.sync_copy(w_hbm.at[pl.ds(base, EPW)], w_v)

    def step(i, carry):
        o = i * L
        s16 = src_v[pl.ds(o, L)]
        d16 = dst_v[pl.ds(o, L)]
        w16 = w_v[pl.ds(o, L)]
        ks = plsc.load_gather(kept_v, [s16])
        kd = plsc.load_gather(kept_v, [d16])
        srcn_v[pl.ds(o, L)] = plsc.load_gather(imap_v, [s16])
        dstn_v[pl.ds(o, L)] = plsc.load_gather(imap_v, [d16])
        wn_v[pl.ds(o, L)] = (w16 * ks) * kd
        return carry

    lax.fori_loop(0, EPW // L, step, 0)
    pltpu.sync_copy(srcn_v, srcn_hbm.at[pl.ds(base, EPW)])
    pltpu.sync_copy(dstn_v, dstn_hbm.at[pl.ds(base, EPW)])
    pltpu.sync_copy(wn_v, wn_hbm.at[pl.ds(base, EPW)])


def sc_remap(src, dst, w, imap, kept):
    m = imap.shape[0]
    mesh = plsc.VectorSubcoreMesh(core_axis_name="c", subcore_axis_name="s")
    f = functools.partial(
        pl.kernel,
        mesh=mesh,
        compiler_params=pltpu.CompilerParams(needs_layout_passes=False),
        out_type=[
            jax.ShapeDtypeStruct((E,), jnp.int32),
            jax.ShapeDtypeStruct((E,), jnp.int32),
            jax.ShapeDtypeStruct((E,), jnp.float32),
        ],
        scratch_types=[
            pltpu.VMEM((m,), jnp.int32),
            pltpu.VMEM((m,), jnp.float32),
            pltpu.VMEM((EPW,), jnp.int32),
            pltpu.VMEM((EPW,), jnp.int32),
            pltpu.VMEM((EPW,), jnp.float32),
            pltpu.VMEM((EPW,), jnp.int32),
            pltpu.VMEM((EPW,), jnp.int32),
            pltpu.VMEM((EPW,), jnp.float32),
        ],
    )(_remap_body)
    return f(src, dst, w, imap, kept)


def _edgeval_body(src_hbm, dst_hbm, w_hbm, dinv_hbm, h_hbm,
                  val_hbm,
                  dinv_v, h_v, src_v, dst_v, w_v, val_v):
    base = _wid() * EPW
    pltpu.sync_copy(dinv_hbm, dinv_v)
    pltpu.sync_copy(h_hbm, h_v)
    pltpu.sync_copy(src_hbm.at[pl.ds(base, EPW)], src_v)
    pltpu.sync_copy(dst_hbm.at[pl.ds(base, EPW)], dst_v)
    pltpu.sync_copy(w_hbm.at[pl.ds(base, EPW)], w_v)

    def step(i, carry):
        o = i * L
        s16 = src_v[pl.ds(o, L)]
        d16 = dst_v[pl.ds(o, L)]
        w16 = w_v[pl.ds(o, L)]
        dis = plsc.load_gather(dinv_v, [s16])
        did = plsc.load_gather(dinv_v, [d16])
        hs = plsc.load_gather(h_v, [s16])
        val_v[pl.ds(o, L)] = ((dis * did) * w16) * hs
        return carry

    lax.fori_loop(0, EPW // L, step, 0)
    pltpu.sync_copy(val_v, val_hbm.at[pl.ds(base, EPW)])


def sc_edge_vals(src, dst, w, dinv, h):
    m = dinv.shape[0]
    mesh = plsc.VectorSubcoreMesh(core_axis_name="c", subcore_axis_name="s")
    f = functools.partial(
        pl.kernel,
        mesh=mesh,
        compiler_params=pltpu.CompilerParams(needs_layout_passes=False),
        out_type=jax.ShapeDtypeStruct((E,), jnp.float32),
        scratch_types=[
            pltpu.VMEM((m,), jnp.float32),
            pltpu.VMEM((m,), jnp.float32),
            pltpu.VMEM((EPW,), jnp.int32),
            pltpu.VMEM((EPW,), jnp.int32),
            pltpu.VMEM((EPW,), jnp.float32),
            pltpu.VMEM((EPW,), jnp.float32),
        ],
    )(_edgeval_body)
    return f(src, dst, w, dinv, h)


_R = 5120
_SRED = 640


def _deg_body(passes, mpad, dst_hbm, w_hbm, out_hbm,
              dst_v, w_v, hist_v, red_v, acc_v, tmp_v, shared_v):
    cid = lax.axis_index("c")
    sid = lax.axis_index("s")
    wid = sid * NC + cid
    base_e = wid * EPW
    pltpu.sync_copy(dst_hbm.at[pl.ds(base_e, EPW)], dst_v)
    pltpu.sync_copy(w_hbm.at[pl.ds(base_e, EPW)], w_v)
    lane = lax.iota(jnp.int32, L)
    zero16 = jnp.zeros((L,), jnp.float32)

    for p in range(passes):
        base = p * _R

        def zstep(i, c):
            hist_v[pl.ds(i * L, L)] = zero16
            return c

        lax.fori_loop(0, (L * _R) // L, zstep, 0)

        def estep(i, c):
            o = i * L
            d16 = dst_v[pl.ds(o, L)]
            w16 = w_v[pl.ds(o, L)]
            rel = d16 - base
            msk = (rel >= 0) & (rel < _R)
            idx = lane * _R + jnp.where(msk, rel, 0)
            plsc.addupdate_scatter(hist_v, [idx], w16, mask=msk)
            return c

        lax.fori_loop(0, EPW // L, estep, 0)

        def rstep(i, c):
            o = i * L
            acc = hist_v[pl.ds(o, L)]
            for ln in range(1, L):
                acc = acc + hist_v[pl.ds(ln * _R + o, L)]
            red_v[pl.ds(o, L)] = acc
            return c

        lax.fori_loop(0, _R // L, rstep, 0)
        pltpu.sync_copy(red_v, shared_v.at[pl.ds(sid * mpad + base, _R)])

    plsc.subcore_barrier()
    nred = mpad // _SRED
    off = sid * _SRED

    @pl.when(sid < nred)
    def _reduce():
        def z2(i, c):
            acc_v[pl.ds(i * L, L)] = zero16
            return c

        lax.fori_loop(0, _SRED // L, z2, 0)
        for t in range(NS):
            pltpu.sync_copy(shared_v.at[pl.ds(t * mpad + off, _SRED)], tmp_v)

            def astep(i, c):
                o = i * L
                acc_v[pl.ds(o, L)] = acc_v[pl.ds(o, L)] + tmp_v[pl.ds(o, L)]
                return c

            lax.fori_loop(0, _SRED // L, astep, 0)
        pltpu.sync_copy(acc_v, out_hbm.at[pl.ds(cid * mpad + off, _SRED)])


def sc_deg(dst, w, m):
    passes = -(-m // _R)
    mpad = passes * _R
    mesh = plsc.VectorSubcoreMesh(core_axis_name="c", subcore_axis_name="s")
    f = functools.partial(
        pl.kernel,
        mesh=mesh,
        compiler_params=pltpu.CompilerParams(needs_layout_passes=False),
        out_type=jax.ShapeDtypeStruct((NC * mpad,), jnp.float32),
        scratch_types=[
            pltpu.VMEM((EPW,), jnp.int32),
            pltpu.VMEM((EPW,), jnp.float32),
            pltpu.VMEM((L * _R,), jnp.float32),
            pltpu.VMEM((_R,), jnp.float32),
            pltpu.VMEM((_SRED,), jnp.float32),
            pltpu.VMEM((_SRED,), jnp.float32),
            pltpu.VMEM_SHARED((NS * mpad,), jnp.float32),
        ],
    )(functools.partial(_deg_body, passes, mpad))
    part = f(dst, w)
    return (part[:mpad] + part[mpad:])[:m]


def _topk_perm(score, k):
    i = lax.bitcast_convert_type(score, jnp.int32)
    key = jnp.where(i < 0, i ^ jnp.int32(0x7FFFFFFF), i)
    order = jnp.lexsort((jnp.arange(score.shape[0], dtype=jnp.int32), ~key))
    return order[:k].astype(jnp.int32)


def _score(h, src, dst, w, dinv, b):
    vals = sc_edge_vals(src, dst, w, dinv, h[:, 0])
    agg = jnp.zeros_like(h).at[dst].add(vals[:, None])
    agg = agg + (dinv * dinv)[:, None] * h
    return (agg + b)[:, 0]


def _pool(xh, score, k, n_nodes):
    perm = _topk_perm(score, k)
    x_new = xh[perm] * p_tanh(score[perm])[:, None]
    kept = jnp.zeros((n_nodes,), jnp.float32).at[perm].set(1.0)
    imap = jnp.zeros((n_nodes,), jnp.int32).at[perm].set(
        jnp.arange(k, dtype=jnp.int32))
    return perm, x_new, kept, imap


def _readout(x):
    return jnp.concatenate([jnp.max(x, axis=0), jnp.mean(x, axis=0)])[None, :]


def kernel(x, edge_index, batch, W1, b1, Ws1, bs1, W2, b2, Ws2, bs2, W3, b3, Ws3, bs3, L1W, L1b, L2W, L2b, L3W, L3b):
    src, dst = edge_index[0], edge_index[1]
    w = jnp.ones((E,), jnp.float32)

    xh, h1 = mm_relu_h(x, W1, b1, Ws1, block=1000)
    deg1 = sc_deg(dst, w, N)
    dinv1 = p_dinv(deg1)
    score1 = _score(h1, src, dst, w, dinv1, bs1)
    perm1, x_p1, kept1, imap1 = _pool(xh, score1, K1, N)
    x1 = _readout(x_p1)

    src2, dst2, w2 = sc_remap(src, dst, w, imap1, kept1)
    x2h, h2 = mm_relu_h(x_p1, W2, b2, Ws2, block=1000)
    deg2 = sc_deg(dst2, w2, K1)
    dinv2 = p_dinv(deg2)
    score2 = _score(h2, src2, dst2, w2, dinv2, bs2)
    perm2, x_p2, kept2, imap2 = _pool(x2h, score2, K2, K1)
    x2 = _readout(x_p2)

    src3, dst3, w3 = sc_remap(src2, dst2, w2, imap2, kept2)
    x3h, h3 = mm_relu_h(x_p2, W3, b3, Ws3, block=2500)
    deg3 = sc_deg(dst3, w3, K2)
    dinv3 = p_dinv(deg3)
    score3 = _score(h3, src3, dst3, w3, dinv3, bs3)
    perm3 = _topk_perm(score3, K3)
    x_p3 = x3h[perm3] * p_tanh(score3[perm3])[:, None]
    x3 = _readout(x_p3)

    xo = x1 + x2 + x3

    def head(v):
        v = jax.nn.relu(v @ L1W + L1b)
        v = jax.nn.relu(v @ L2W + L2b)
        return jax.nn.log_softmax(v @ L3W + L3b, axis=-1)

    ho = head(xo)
    return (ho, ho, score1, score1, score2, score2, score3, score3)
